# baseline (speedup 1.0000x reference)
import functools

import jax
import jax.numpy as jnp
from jax import lax
from jax.experimental import pallas as pl
from jax.experimental.pallas import tpu as pltpu
from jax.experimental.pallas import tpu_sc as plsc

_SUB = 128
_CHUNK = 512


def _gather_body(n_chunks, num_cores, tok_hbm, w_hbm, out_hbm, idx_v, rows_v, sem):
    k = _CHUNK // _SUB
    wid = lax.axis_index("s") * num_cores + lax.axis_index("c")
    w_row_base = wid * (n_chunks * k)
    w_base = wid * (n_chunks * _CHUNK)

    def body(i, carry):
        row0 = w_row_base + i * k
        base = w_base + i * _CHUNK
        pltpu.sync_copy(tok_hbm.at[pl.ds(row0, k)], idx_v)
        copies = [
            pltpu.async_copy(
                w_hbm.at[idx_v.at[j]], rows_v.at[pl.ds(j * _SUB, _SUB)], sem
            )
            for j in range(k)
        ]
        for cp in copies:
            cp.wait()
        pltpu.sync_copy(rows_v, out_hbm.at[pl.ds(base, _CHUNK)])
        return carry

    lax.fori_loop(0, n_chunks, body, 0)


def kernel(token_ids, W):
    B = token_ids.size
    D = W.shape[1]
    info = plsc.get_sparse_core_info()
    nw = info.num_cores * info.num_subcores
    n_chunks = B // (nw * _CHUNK)
    tok2d = token_ids.reshape(B // _SUB, _SUB)

    mesh = plsc.VectorSubcoreMesh(core_axis_name="c", subcore_axis_name="s")
    kfn = pl.kernel(
        functools.partial(_gather_body, n_chunks, info.num_cores),
        out_type=jax.ShapeDtypeStruct((B, D), jnp.float32),
        mesh=mesh,
        scratch_types=[
            pltpu.VMEM((_CHUNK // _SUB, _SUB), jnp.int32),
            pltpu.VMEM((_CHUNK, D), jnp.float32),
            pltpu.SemaphoreType.DMA,
        ],
        compiler_params=pltpu.CompilerParams(use_tc_tiling_on_sc=False),
    )
    out = kfn(tok2d, W)
    return out.reshape(*token_ids.shape, D)


# speedup vs baseline: 1.0480x; 1.0480x over previous
import functools

import jax
import jax.numpy as jnp
from jax import lax
from jax.experimental import pallas as pl
from jax.experimental.pallas import tpu as pltpu
from jax.experimental.pallas import tpu_sc as plsc

_SUB = 128
_CHUNK = 640
_K = _CHUNK // _SUB


def _gather_body(
    n_chunks, num_cores, tok_hbm, w_hbm, out_hbm, idx_v, rows_v, sem_idx, sem_g, sem_o
):
    wid = lax.axis_index("s") * num_cores + lax.axis_index("c")
    row_base = wid * (n_chunks * _K)
    out_base = wid * (n_chunks * _CHUNK)

    def idx_start(i, slot):
        pltpu.async_copy(
            tok_hbm.at[pl.ds(row_base + i * _K, _K)], idx_v.at[slot], sem_idx
        )

    def idx_wait(slot):
        pltpu.make_async_copy(
            tok_hbm.at[pl.ds(row_base, _K)], idx_v.at[slot], sem_idx
        ).wait()

    def gathers_start(slot):
        for j in range(_K):
            pltpu.async_copy(
                w_hbm.at[idx_v.at[slot, j]],
                rows_v.at[slot].at[pl.ds(j * _SUB, _SUB)],
                sem_g,
            )

    def gathers_wait(slot):
        for j in range(_K):
            pltpu.make_async_copy(
                w_hbm.at[pl.ds(0, _SUB)],
                rows_v.at[slot].at[pl.ds(j * _SUB, _SUB)],
                sem_g,
            ).wait()

    def out_start(i, slot):
        pltpu.async_copy(
            rows_v.at[slot], out_hbm.at[pl.ds(out_base + i * _CHUNK, _CHUNK)], sem_o
        )

    def out_wait(slot):
        pltpu.make_async_copy(
            rows_v.at[slot], out_hbm.at[pl.ds(out_base, _CHUNK)], sem_o
        ).wait()

    idx_start(0, 0)
    idx_wait(0)
    gathers_start(0)
    idx_start(1, 1)

    def loop_body(i, carry):
        slot = lax.rem(i, 2)
        nslot = lax.rem(i + 1, 2)
        gathers_wait(slot)
        out_start(i, slot)
        idx_wait(nslot)

        @pl.when(i >= 1)
        def _():
            out_wait(nslot)

        gathers_start(nslot)

        @pl.when(i + 2 < n_chunks)
        def _():
            idx_start(i + 2, slot)

        return carry

    lax.fori_loop(0, n_chunks - 1, loop_body, 0)

    last_slot = (n_chunks - 1) % 2
    gathers_wait(last_slot)
    out_start(n_chunks - 1, last_slot)
    out_wait(1 - last_slot)
    out_wait(last_slot)


def kernel(token_ids, W):
    B = token_ids.size
    D = W.shape[1]
    info = plsc.get_sparse_core_info()
    nw = info.num_cores * info.num_subcores
    n_chunks = B // (nw * _CHUNK)
    tok2d = token_ids.reshape(B // _SUB, _SUB)

    mesh = plsc.VectorSubcoreMesh(core_axis_name="c", subcore_axis_name="s")
    kfn = pl.kernel(
        functools.partial(_gather_body, n_chunks, info.num_cores),
        out_type=jax.ShapeDtypeStruct((B, D), jnp.float32),
        mesh=mesh,
        scratch_types=[
            pltpu.VMEM((2, _K, _SUB), jnp.int32),
            pltpu.VMEM((2, _CHUNK, D), jnp.float32),
            pltpu.SemaphoreType.DMA,
            pltpu.SemaphoreType.DMA,
            pltpu.SemaphoreType.DMA,
        ],
        compiler_params=pltpu.CompilerParams(use_tc_tiling_on_sc=False),
    )
    out = kfn(tok2d, W)
    return out.reshape(*token_ids.shape, D)
